# baseline (speedup 1.0000x reference)
import functools

import jax
import jax.numpy as jnp
from jax import lax
from jax.experimental import pallas as pl
from jax.experimental.pallas import tpu as pltpu
from jax.experimental.pallas import tpu_sc as plsc

MAX_POS_ = 2048
HIDDEN_ = 768
BATCH_ = 4

NUM_CORES = 2
NUM_SUBCORES = 16
NUM_WORKERS = NUM_CORES * NUM_SUBCORES
P_PER_W = MAX_POS_ // NUM_WORKERS
WORDS_PER_W = P_PER_W * HIDDEN_
LANES = 16
NCHUNK = WORDS_PER_W // LANES


def _sc_body(hid_hbm, pos_hbm, out_hbm, pos_v, hid_v):
    wid = lax.axis_index("s") * NUM_CORES + lax.axis_index("c")
    base = wid * WORDS_PER_W
    pltpu.sync_copy(pos_hbm.at[pl.ds(base, WORDS_PER_W)], pos_v)

    def add_chunk(i, carry):
        off = i * LANES
        hid_v[pl.ds(off, LANES)] = hid_v[pl.ds(off, LANES)] + pos_v[pl.ds(off, LANES)]
        return carry

    for b in range(BATCH_):
        pltpu.sync_copy(hid_hbm.at[b].at[pl.ds(base, WORDS_PER_W)], hid_v)
        lax.fori_loop(0, NCHUNK, add_chunk, 0)
        pltpu.sync_copy(hid_v, out_hbm.at[b].at[pl.ds(base, WORDS_PER_W)])


def _sc_call(hid_flat, pos_flat):
    mesh = plsc.VectorSubcoreMesh(core_axis_name="c", subcore_axis_name="s")
    run = functools.partial(
        pl.kernel,
        mesh=mesh,
        out_type=jax.ShapeDtypeStruct((BATCH_, MAX_POS_ * HIDDEN_), jnp.float32),
        scratch_types=[
            pltpu.VMEM((WORDS_PER_W,), jnp.float32),
            pltpu.VMEM((WORDS_PER_W,), jnp.float32),
        ],
    )(_sc_body)
    return run(hid_flat, pos_flat)


def kernel(hidden_states, pos_table):
    hid_flat = hidden_states.reshape(BATCH_, MAX_POS_ * HIDDEN_)
    pos_flat = pos_table.reshape(MAX_POS_ * HIDDEN_)
    out = _sc_call(hid_flat, pos_flat)
    return out.reshape(BATCH_, MAX_POS_, HIDDEN_)


# speedup vs baseline: 1.6090x; 1.6090x over previous
import functools

import jax
import jax.numpy as jnp
from jax import lax
from jax.experimental import pallas as pl
from jax.experimental.pallas import tpu as pltpu
from jax.experimental.pallas import tpu_sc as plsc

MAX_POS_ = 2048
HIDDEN_ = 768
BATCH_ = 4

NUM_CORES = 2
NUM_SUBCORES = 16
NUM_WORKERS = NUM_CORES * NUM_SUBCORES
P_PER_W = MAX_POS_ // NUM_WORKERS
WORDS_PER_W = P_PER_W * HIDDEN_
LANES = 16

NHALF = 2
CH_WORDS = WORDS_PER_W // NHALF
NBUF = 3
NSTEP = NHALF * BATCH_


def _sc_body(hid_hbm, pos_hbm, out_hbm,
             pos0, pos1, buf0, buf1, buf2,
             psem0, psem1, isem0, isem1, isem2, osem0, osem1, osem2):
    wid = lax.axis_index("s") * NUM_CORES + lax.axis_index("c")
    base = wid * WORDS_PER_W

    pos_bufs = [pos0, pos1]
    bufs = [buf0, buf1, buf2]
    psems = [psem0, psem1]
    isems = [isem0, isem1, isem2]
    osems = [osem0, osem1, osem2]

    steps = [(h, b) for h in range(NHALF) for b in range(BATCH_)]

    def hbm_slice(ref, h):
        return ref.at[pl.ds(base + h * CH_WORDS, CH_WORDS)]

    pos_h = [
        pltpu.async_copy(hbm_slice(pos_hbm, h), pos_bufs[h], psems[h])
        for h in range(NHALF)
    ]

    in_h = [None] * NSTEP
    out_h = [None] * NSTEP
    for s in range(NBUF):
        h, b = steps[s]
        in_h[s] = pltpu.async_copy(
            hbm_slice(hid_hbm.at[b], h), bufs[s % NBUF], isems[s % NBUF])

    for s in range(NSTEP):
        h, b = steps[s]
        if h * BATCH_ == s:
            pos_h[h].wait()
        in_h[s].wait()
        hid_buf = bufs[s % NBUF]
        pos_buf = pos_bufs[h]

        @plsc.parallel_loop(0, CH_WORDS, step=LANES, unroll=8)
        def _(off):
            plsc.addupdate(hid_buf.at[pl.ds(off, LANES)],
                           pos_buf[pl.ds(off, LANES)])

        out_h[s] = pltpu.async_copy(
            hid_buf, hbm_slice(out_hbm.at[b], h), osems[s % NBUF])

        ns = s + 2
        if ns < NSTEP and in_h[ns] is None:
            if s >= 1:
                out_h[s - 1].wait()
            nh, nb = steps[ns]
            in_h[ns] = pltpu.async_copy(
                hbm_slice(hid_hbm.at[nb], nh), bufs[ns % NBUF], isems[ns % NBUF])

    out_h[NSTEP - 2].wait()
    out_h[NSTEP - 1].wait()


def _sc_call(hid_flat, pos_flat):
    mesh = plsc.VectorSubcoreMesh(core_axis_name="c", subcore_axis_name="s")
    run = functools.partial(
        pl.kernel,
        mesh=mesh,
        out_type=jax.ShapeDtypeStruct((BATCH_, MAX_POS_ * HIDDEN_), jnp.float32),
        scratch_types=(
            [pltpu.VMEM((CH_WORDS,), jnp.float32) for _ in range(NHALF + NBUF)]
            + [pltpu.SemaphoreType.DMA for _ in range(NHALF + 2 * NBUF)]
        ),
    )(_sc_body)
    return run(hid_flat, pos_flat)


def kernel(hidden_states, pos_table):
    hid_flat = hidden_states.reshape(BATCH_, MAX_POS_ * HIDDEN_)
    pos_flat = pos_table.reshape(MAX_POS_ * HIDDEN_)
    out = _sc_call(hid_flat, pos_flat)
    return out.reshape(BATCH_, MAX_POS_, HIDDEN_)


# speedup vs baseline: 3.0383x; 1.8883x over previous
import functools

import jax
import jax.numpy as jnp
from jax import lax
from jax.experimental import pallas as pl
from jax.experimental.pallas import tpu as pltpu
from jax.experimental.pallas import tpu_sc as plsc

MAX_POS_ = 2048
HIDDEN_ = 768
BATCH_ = 4

NUM_CORES = 2
NUM_SUBCORES = 16
NUM_WORKERS = NUM_CORES * NUM_SUBCORES
P_PER_W = MAX_POS_ // NUM_WORKERS
LANES = 16

NHALF = 2
CH_ROWS = P_PER_W // NHALF
NBUF = 3
NSTEP = NHALF * BATCH_


def _sc_body(hid_hbm, pos_hbm, out_hbm,
             pos0, pos1, buf0, buf1, buf2,
             psem0, psem1, isem0, isem1, isem2, osem0, osem1, osem2):
    wid = lax.axis_index("s") * NUM_CORES + lax.axis_index("c")
    row_base = wid * P_PER_W

    pos_bufs = [pos0, pos1]
    bufs = [buf0, buf1, buf2]
    psems = [psem0, psem1]
    isems = [isem0, isem1, isem2]
    osems = [osem0, osem1, osem2]

    steps = [(h, b) for h in range(NHALF) for b in range(BATCH_)]

    def rows(ref, h):
        return ref.at[pl.ds(row_base + h * CH_ROWS, CH_ROWS)]

    pos_h = [
        pltpu.async_copy(rows(pos_hbm, h), pos_bufs[h], psems[h])
        for h in range(NHALF)
    ]

    in_h = [None] * NSTEP
    out_h = [None] * NSTEP
    for s in range(NBUF):
        h, b = steps[s]
        in_h[s] = pltpu.async_copy(
            rows(hid_hbm.at[b], h), bufs[s % NBUF], isems[s % NBUF])

    for s in range(NSTEP):
        h, b = steps[s]
        if h * BATCH_ == s:
            pos_h[h].wait()
        in_h[s].wait()
        hid_buf = bufs[s % NBUF]
        pos_buf = pos_bufs[h]

        @plsc.parallel_loop(0, CH_ROWS, step=1)
        def _(r):
            for c in range(0, HIDDEN_, LANES):
                plsc.addupdate(hid_buf.at[r, pl.ds(c, LANES)],
                               pos_buf[r, pl.ds(c, LANES)])

        out_h[s] = pltpu.async_copy(
            hid_buf, rows(out_hbm.at[b], h), osems[s % NBUF])

        ns = s + 2
        if ns < NSTEP and in_h[ns] is None:
            if s >= 1:
                out_h[s - 1].wait()
            nh, nb = steps[ns]
            in_h[ns] = pltpu.async_copy(
                rows(hid_hbm.at[nb], nh), bufs[ns % NBUF], isems[ns % NBUF])

    out_h[NSTEP - 2].wait()
    out_h[NSTEP - 1].wait()


def kernel(hidden_states, pos_table):
    mesh = plsc.VectorSubcoreMesh(core_axis_name="c", subcore_axis_name="s")
    run = functools.partial(
        pl.kernel,
        mesh=mesh,
        out_type=jax.ShapeDtypeStruct((BATCH_, MAX_POS_, HIDDEN_), jnp.float32),
        scratch_types=(
            [pltpu.VMEM((CH_ROWS, HIDDEN_), jnp.float32)
             for _ in range(NHALF + NBUF)]
            + [pltpu.SemaphoreType.DMA for _ in range(NHALF + 2 * NBUF)]
        ),
        compiler_params=pltpu.CompilerParams(use_tc_tiling_on_sc=True),
    )(_sc_body)
    return run(hidden_states, pos_table)


# speedup vs baseline: 7.0511x; 2.3208x over previous
import jax
import jax.numpy as jnp
from jax.experimental import pallas as pl

MAX_POS_ = 2048
HIDDEN_ = 768
BATCH_ = 4

BP = 1024


def _add_body(hid_ref, pos_ref, out_ref):
    out_ref[...] = hid_ref[...] + pos_ref[...]


def kernel(hidden_states, pos_table):
    n_pos_blocks = MAX_POS_ // BP
    grid = (n_pos_blocks, BATCH_)
    return pl.pallas_call(
        _add_body,
        grid=grid,
        in_specs=[
            pl.BlockSpec((1, BP, HIDDEN_), lambda i, b: (b, i, 0)),
            pl.BlockSpec((BP, HIDDEN_), lambda i, b: (i, 0)),
        ],
        out_specs=pl.BlockSpec((1, BP, HIDDEN_), lambda i, b: (b, i, 0)),
        out_shape=jax.ShapeDtypeStruct((BATCH_, MAX_POS_, HIDDEN_), jnp.float32),
    )(hidden_states, pos_table)


# speedup vs baseline: 7.5882x; 1.0762x over previous
import jax
import jax.numpy as jnp
from jax.experimental import pallas as pl

MAX_POS_ = 2048
HIDDEN_ = 768
BATCH_ = 4

BP = 2048


def _add_body(hid_ref, pos_ref, out_ref):
    out_ref[...] = hid_ref[...] + pos_ref[...]


def kernel(hidden_states, pos_table):
    n_pos_blocks = MAX_POS_ // BP
    grid = (n_pos_blocks, BATCH_)
    return pl.pallas_call(
        _add_body,
        grid=grid,
        in_specs=[
            pl.BlockSpec((1, BP, HIDDEN_), lambda i, b: (b, i, 0)),
            pl.BlockSpec((BP, HIDDEN_), lambda i, b: (i, 0)),
        ],
        out_specs=pl.BlockSpec((1, BP, HIDDEN_), lambda i, b: (b, i, 0)),
        out_shape=jax.ShapeDtypeStruct((BATCH_, MAX_POS_, HIDDEN_), jnp.float32),
    )(hidden_states, pos_table)
